# baseline (speedup 1.0000x reference)
import jax
import jax.numpy as jnp
from jax import lax
from jax.experimental import pallas as pl
from jax.experimental.pallas import tpu as pltpu
from jax.experimental.pallas import tpu_sc as plsc

N_FACTORS = 64
VROW = 128
BATCH = 16384

NC = 2
NS = 16
L = 16
NW = NC * NS
B_PER_W = BATCH // NW
CHUNK = 128
NCHUNK = B_PER_W // CHUNK
GROUPS = CHUNK // L


def _sc_body(uids_hbm, mids_hbm, ulin_hbm, mlin_hbm, out_hbm,
             uidx_v, midx_v, urow_idx_v, mrow_idx_v,
             urows_v, mrows_v, outv, sems):
    wid = lax.axis_index("s") * NC + lax.axis_index("c")
    base = wid * B_PER_W

    for c in range(NCHUNK):
        pltpu.sync_copy(uids_hbm.at[pl.ds(base + c * CHUNK, CHUNK)],
                        uidx_v.at[c])
        pltpu.sync_copy(mids_hbm.at[pl.ds(base + c * CHUNK, CHUNK)],
                        midx_v.at[c])

    for c in range(NCHUNK):
        for v in range(CHUNK // L):
            s = pl.ds(v * L, L)
            urow_idx_v[c, s] = lax.shift_right_logical(uidx_v[c, s], 1)
            mrow_idx_v[c, s] = lax.shift_right_logical(midx_v[c, s], 1)

    lane = lax.iota(jnp.int32, L)

    def fire(c):
        buf = c % 2
        return (pltpu.async_copy(ulin_hbm.at[urow_idx_v.at[c]],
                                 urows_v.at[buf], sems.at[buf]),
                pltpu.async_copy(mlin_hbm.at[mrow_idx_v.at[c]],
                                 mrows_v.at[buf], sems.at[buf]))

    def compute_chunk(c, buf):
        for g in range(GROUPS):
            rvec = g * L + lane
            upar = (uidx_v[c, pl.ds(g * L, L)] & 1) * N_FACTORS
            mpar = (midx_v[c, pl.ds(g * L, L)] & 1) * N_FACTORS

            def dstep(d, acc):
                dvec = (d + lane) & (N_FACTORS - 1)
                u = plsc.load_gather(urows_v.at[buf], [rvec, upar + dvec])
                m = plsc.load_gather(mrows_v.at[buf], [rvec, mpar + dvec])
                return acc + u * m

            acc = lax.fori_loop(0, N_FACTORS, dstep,
                                jnp.zeros((L,), jnp.float32))
            outv[pl.ds(c * CHUNK + g * L, L)] = acc

    pending = fire(0)
    for c in range(NCHUNK):
        for cp in pending:
            cp.wait()
        if c + 1 < NCHUNK:
            pending = fire(c + 1)
        compute_chunk(c, c % 2)

    pltpu.sync_copy(outv, out_hbm.at[pl.ds(base, B_PER_W)])


@jax.jit
def _mf_dot(user_ids, movie_ids, user_lin, movie_lin):
    mesh = plsc.VectorSubcoreMesh(core_axis_name="c", subcore_axis_name="s")
    kfn = pl.kernel(
        _sc_body,
        mesh=mesh,
        out_type=jax.ShapeDtypeStruct((BATCH,), jnp.float32),
        compiler_params=pltpu.CompilerParams(needs_layout_passes=False),
        scratch_types=[
            pltpu.VMEM((NCHUNK, CHUNK), jnp.int32),
            pltpu.VMEM((NCHUNK, CHUNK), jnp.int32),
            pltpu.VMEM((NCHUNK, CHUNK), jnp.int32),
            pltpu.VMEM((NCHUNK, CHUNK), jnp.int32),
            pltpu.VMEM((2, CHUNK, VROW), jnp.float32),
            pltpu.VMEM((2, CHUNK, VROW), jnp.float32),
            pltpu.VMEM((B_PER_W,), jnp.float32),
            pltpu.SemaphoreType.DMA((2,)),
        ],
    )
    return kfn(user_ids, movie_ids, user_lin, movie_lin)


def kernel(user_ids, movie_ids, user_emb, movie_emb):
    nu, nm = user_emb.shape[0], movie_emb.shape[0]
    return _mf_dot(user_ids.astype(jnp.int32), movie_ids.astype(jnp.int32),
                   user_emb.reshape(nu // 2, VROW),
                   movie_emb.reshape(nm // 2, VROW))
